# Initial kernel scaffold; baseline (speedup 1.0000x reference)
import jax
import jax.numpy as jnp
from jax.experimental import pallas as pl


def kernel(points):
    raise NotImplementedError("write your pallas kernel here")


# speedup vs baseline: 2.8057x; 2.8057x over previous
import functools

import jax
import jax.numpy as jnp
from jax import lax
from jax.experimental import pallas as pl
from jax.experimental.pallas import tpu as pltpu
from jax.experimental.pallas import tpu_sc as plsc

N = 20000
NT = 16
CH = 1280
NPAD = NT * CH
RB = 1248
NCH = 10
MAXKEY = 0x7FFFFFFF
C1 = (1 << 30) - 1
C2 = 1 << 30
CNT_BINS = 8448
STRIPE = CNT_BINS // NT


def _body(pts_hbm, out_hbm,
          keyA, idxA, keyB, idxB, ckey_sh, pts_sh, hist_grid, cnt8k, rank8k,
          stats_grid,
          ptbuf, outbuf, karr, iarr, ckarr, ordarr, kv, iv, posb, sidxb,
          cksb, rksb, gb, jbf, idx4, grid_l, h512, rk512,
          ones128, zstripe, hist256, baserow, sem):
  t = lax.axis_index("s")
  lanes = lax.iota(jnp.int32, 16)
  zeros16 = jnp.zeros((16,), jnp.int32)
  col1 = jnp.full((16,), 1, jnp.int32)
  col2 = jnp.full((16,), 2, jnp.int32)
  col3 = jnp.full((16,), 3, jnp.int32)

  base_row = RB * t
  pbase = CH * t
  nreal = jnp.where(t == NT - 1, jnp.int32(CH), jnp.int32(RB))

  pltpu.sync_copy(pts_hbm.at[pl.ds(base_row * 4, CH * 4)], ptbuf)

  def _ob(i, c):
    ones128[pl.ds(i * 16, 16)] = jnp.full((16,), 1, jnp.int32)
    return c

  lax.fori_loop(0, 8, _ob, jnp.int32(0))

  def _zb(i, c):
    zstripe[pl.ds(i * 16, 16)] = zeros16
    return c

  lax.fori_loop(0, STRIPE // 16, _zb, jnp.int32(0))
  pltpu.sync_copy(zstripe, cnt8k.at[pl.ds(STRIPE * t, STRIPE)])
  plsc.subcore_barrier()

  def _s0(c, carry):
    for k in range(8):
      sl = pl.ds(16 * k, 16)
      e = 128 * c + 16 * k + lanes
      e4 = e * 4
      x = plsc.load_gather(ptbuf, [e4])
      y = plsc.load_gather(ptbuf, [e4 + 1])
      s = plsc.load_gather(ptbuf, [e4 + 2])
      f = plsc.load_gather(ptbuf, [e4 + 3])
      xi = x.astype(jnp.int32)
      yi = y.astype(jnp.int32)
      fi = f.astype(jnp.int32)
      sb = plsc.bitcast(s, jnp.int32)
      valid = e < nreal
      key = jnp.where(fi == 1, C1 - sb, C2 + sb)
      key = jnp.where(valid, key, MAXKEY)
      ck = jnp.where(fi == 1, jnp.int32(0), jnp.int32(4096)) + xi * 64 + yi
      ck = jnp.where(valid, ck, 8192 + (e & 255))
      karr[c, sl] = key
      iarr[c, sl] = pbase + e
      ckarr[c, sl] = ck
      plsc.store_scatter(ptbuf, [e4], x * 16.0)
      plsc.store_scatter(ptbuf, [e4 + 1], y * 16.0)
    pltpu.sync_copy(karr.at[c], keyA.at[pl.ds(pbase + 128 * c, 128)])
    pltpu.sync_copy(iarr.at[c], idxA.at[pl.ds(pbase + 128 * c, 128)])
    pltpu.sync_copy(ckarr.at[c], ckey_sh.at[pl.ds(pbase + 128 * c, 128)])
    pltpu.async_copy(ones128, cnt8k.at[ckarr.at[c]], sem, add=True).wait()
    return carry

  lax.fori_loop(0, NCH, _s0, jnp.int32(0))
  pltpu.sync_copy(ptbuf, pts_sh.at[pl.ds(pbase * 4, CH * 4)])
  plsc.subcore_barrier()

  def _one_pass(shift, cur_k, cur_i, nxt_k, nxt_i):
    pltpu.sync_copy(cur_k.at[pl.ds(pbase, CH)], kv)
    pltpu.sync_copy(cur_i.at[pl.ds(pbase, CH)], iv)

    def _zh(i, c):
      hist256[pl.ds(i * 16, 16)] = zeros16
      return c

    lax.fori_loop(0, 16, _zh, jnp.int32(0))

    def _ha(c, carry):
      for k in range(8):
        sl = pl.ds(128 * c + 16 * k, 16)
        kk = kv[sl]
        d = (kk >> shift) & 255
        base = plsc.load_gather(hist256, [d])
        cnt, last = plsc.scan_count(d)
        ordarr[sl] = base + cnt - 1
        plsc.store_scatter(hist256, [d], base + cnt, mask=last)
      return carry

    lax.fori_loop(0, NCH, _ha, jnp.int32(0))
    pltpu.sync_copy(hist256, hist_grid.at[t])
    plsc.subcore_barrier()
    pltpu.sync_copy(hist_grid, grid_l)

    def _bs(c16, carry):
      tot = zeros16
      below = zeros16
      for tt in range(NT):
        row = grid_l[tt, pl.ds(16 * c16, 16)]
        tot = tot + row
        below = below + jnp.where(jnp.int32(tt) < t, row, zeros16)
      csum = plsc.cumsum(tot)
      baserow[pl.ds(16 * c16, 16)] = carry + (csum - tot) + below
      return carry + jnp.sum(tot)

    lax.fori_loop(0, 16, _bs, jnp.int32(0))

    def _sc(c, carry):
      for k in range(8):
        sl = pl.ds(128 * c + 16 * k, 16)
        kk = kv[sl]
        d = (kk >> shift) & 255
        b = plsc.load_gather(baserow, [d])
        posb[c, pl.ds(16 * k, 16)] = b + ordarr[sl]
      pltpu.async_copy(kv.at[pl.ds(128 * c, 128)], nxt_k.at[posb.at[c]],
                       sem).wait()
      pltpu.async_copy(iv.at[pl.ds(128 * c, 128)], nxt_i.at[posb.at[c]],
                       sem).wait()
      return carry

    lax.fori_loop(0, NCH, _sc, jnp.int32(0))
    plsc.subcore_barrier()

  def _two_passes(p, carry):
    shift = 16 * p
    _one_pass(shift, keyA, idxA, keyB, idxB)
    _one_pass(shift + 8, keyB, idxB, keyA, idxA)
    return carry

  lax.fori_loop(0, 2, _two_passes, jnp.int32(0))

  pltpu.sync_copy(cnt8k.at[pl.ds(512 * t, 512)], h512)

  def _pr(i, accs):
    pacc, cacc = accs
    hv = h512[pl.ds(16 * i, 16)]
    return pacc + (hv > 0).astype(jnp.int32), cacc + hv

  pacc, cacc = lax.fori_loop(
      0, 32, _pr, (jnp.zeros((16,), jnp.int32), jnp.zeros((16,), jnp.int32)))
  hist256[pl.ds(0, 16)] = pacc
  hist256[pl.ds(16, 16)] = cacc
  pltpu.sync_copy(hist256, stats_grid.at[t])
  plsc.subcore_barrier()
  pltpu.sync_copy(stats_grid, grid_l)
  base_t = jnp.int32(0)
  d_fg = jnp.int32(0)
  n_fg = jnp.int32(0)
  for tt in range(NT):
    rs = jnp.sum(grid_l[tt, pl.ds(0, 16)])
    base_t = base_t + jnp.where(jnp.int32(tt) < t, rs, jnp.int32(0))
    if tt < 8:
      cs = jnp.sum(grid_l[tt, pl.ds(16, 16)])
      d_fg = d_fg + rs
      n_fg = n_fg + cs

  def _rk(i, carry):
    pv = (h512[pl.ds(16 * i, 16)] > 0).astype(jnp.int32)
    cs = plsc.cumsum(pv)
    rk512[pl.ds(16 * i, 16)] = carry + cs - 1
    return carry + jnp.sum(pv)

  lax.fori_loop(0, 32, _rk, base_t)
  pltpu.sync_copy(rk512, rank8k.at[pl.ds(512 * t, 512)])
  plsc.subcore_barrier()

  off_bg = n_fg - d_fg
  obase = RB * t

  def _fin(c, carry):
    pltpu.sync_copy(idxA.at[pl.ds(obase + 128 * c, 128)], sidxb.at[c])
    pltpu.async_copy(ckey_sh.at[sidxb.at[c]], cksb.at[c], sem).wait()
    pltpu.async_copy(rank8k.at[cksb.at[c]], rksb.at[c], sem).wait()
    for k in range(8):
      sl = pl.ds(16 * k, 16)
      ck = cksb[c, sl]
      r = rksb[c, sl]
      gb[c, sl] = r + jnp.where(ck >= 4096, off_bg, jnp.int32(0))
    pltpu.async_copy(idxA.at[gb.at[c]], jbf.at[pl.ds(128 * c, 128)],
                     sem).wait()
    for k in range(32):
      w = 512 * c + 16 * k + lanes
      jv = plsc.load_gather(jbf, [w >> 2])
      idx4[4 * c + k // 8, pl.ds((k % 8) * 16, 16)] = jv * 4 + (w & 3)
    for r in range(4):
      pltpu.async_copy(pts_sh.at[idx4.at[4 * c + r]],
                       outbuf.at[pl.ds(512 * c + 128 * r, 128)], sem).wait()
    pltpu.sync_copy(outbuf.at[pl.ds(512 * c, 512)],
                    out_hbm.at[pl.ds((obase + 128 * c) * 4, 512)])
    return carry

  lax.fori_loop(0, NCH, _fin, jnp.int32(0))


_mesh = plsc.VectorSubcoreMesh(core_axis_name="c", subcore_axis_name="s",
                               num_cores=1)

_sc_call = functools.partial(
    pl.kernel,
    out_type=jax.ShapeDtypeStruct((N * 4,), jnp.float32),
    mesh=_mesh,
    compiler_params=pltpu.CompilerParams(needs_layout_passes=False),
    scratch_types=[
        pltpu.VMEM_SHARED((NPAD,), jnp.int32),
        pltpu.VMEM_SHARED((NPAD,), jnp.int32),
        pltpu.VMEM_SHARED((NPAD,), jnp.int32),
        pltpu.VMEM_SHARED((NPAD,), jnp.int32),
        pltpu.VMEM_SHARED((NPAD,), jnp.int32),
        pltpu.VMEM_SHARED((NPAD * 4,), jnp.float32),
        pltpu.VMEM_SHARED((NT, 256), jnp.int32),
        pltpu.VMEM_SHARED((CNT_BINS,), jnp.int32),
        pltpu.VMEM_SHARED((8192,), jnp.int32),
        pltpu.VMEM_SHARED((NT, 256), jnp.int32),
        pltpu.VMEM((CH * 4,), jnp.float32),
        pltpu.VMEM((CH * 4,), jnp.float32),
        pltpu.VMEM((NCH, 128), jnp.int32),
        pltpu.VMEM((NCH, 128), jnp.int32),
        pltpu.VMEM((NCH, 128), jnp.int32),
        pltpu.VMEM((CH,), jnp.int32),
        pltpu.VMEM((CH,), jnp.int32),
        pltpu.VMEM((CH,), jnp.int32),
        pltpu.VMEM((NCH, 128), jnp.int32),
        pltpu.VMEM((NCH, 128), jnp.int32),
        pltpu.VMEM((NCH, 128), jnp.int32),
        pltpu.VMEM((NCH, 128), jnp.int32),
        pltpu.VMEM((NCH, 128), jnp.int32),
        pltpu.VMEM((CH,), jnp.int32),
        pltpu.VMEM((4 * NCH, 128), jnp.int32),
        pltpu.VMEM((NT, 256), jnp.int32),
        pltpu.VMEM((512,), jnp.int32),
        pltpu.VMEM((512,), jnp.int32),
        pltpu.VMEM((128,), jnp.int32),
        pltpu.VMEM((STRIPE,), jnp.int32),
        pltpu.VMEM((256,), jnp.int32),
        pltpu.VMEM((256,), jnp.int32),
        pltpu.SemaphoreType.DMA,
    ],
)(_body)


@jax.jit
def kernel(points):
  flat = jnp.reshape(points.astype(jnp.float32), (N * 4,))
  return jnp.reshape(_sc_call(flat), (N, 4))


# speedup vs baseline: 3.3125x; 1.1807x over previous
import functools

import jax
import jax.numpy as jnp
from jax import lax
from jax.experimental import pallas as pl
from jax.experimental.pallas import tpu as pltpu
from jax.experimental.pallas import tpu_sc as plsc

N = 20000
NT = 16
CH = 1280
NPAD = NT * CH
RB = 1248
NCH = 10
MAXKEY = 0x7FFFFFFF
C1 = (1 << 30) - 1
C2 = 1 << 30
CNT_BINS = 8448
STRIPE = CNT_BINS // NT


def _body(pts_hbm, out_hbm,
          keyA, idxA, keyB, idxB, ckey_sh, pts_sh, hist_grid, cnt8k, rank8k,
          stats_grid,
          ptbuf, outbuf, karr, iarr, ckarr, ordarr, kv, iv, posb, sidxb,
          cksb, rksb, gb, jbf, idx4, grid_l, h512, rk512,
          ones128, zstripe, hist256, baserow, sem):
  t = lax.axis_index("s")
  lanes = lax.iota(jnp.int32, 16)
  zeros16 = jnp.zeros((16,), jnp.int32)
  col1 = jnp.full((16,), 1, jnp.int32)
  col2 = jnp.full((16,), 2, jnp.int32)
  col3 = jnp.full((16,), 3, jnp.int32)

  base_row = RB * t
  pbase = CH * t
  nreal = jnp.where(t == NT - 1, jnp.int32(CH), jnp.int32(RB))

  pltpu.sync_copy(pts_hbm.at[pl.ds(base_row * 4, CH * 4)], ptbuf)

  def _ob(i, c):
    ones128[pl.ds(i * 16, 16)] = jnp.full((16,), 1, jnp.int32)
    return c

  lax.fori_loop(0, 8, _ob, jnp.int32(0))

  def _zb(i, c):
    zstripe[pl.ds(i * 16, 16)] = zeros16
    return c

  lax.fori_loop(0, STRIPE // 16, _zb, jnp.int32(0))
  pltpu.sync_copy(zstripe, cnt8k.at[pl.ds(STRIPE * t, STRIPE)])
  plsc.subcore_barrier()

  def _s0(c, carry):
    for k in range(8):
      sl = pl.ds(16 * k, 16)
      e = 128 * c + 16 * k + lanes
      e4 = e * 4
      x = plsc.load_gather(ptbuf, [e4])
      y = plsc.load_gather(ptbuf, [e4 + 1])
      s = plsc.load_gather(ptbuf, [e4 + 2])
      f = plsc.load_gather(ptbuf, [e4 + 3])
      xi = x.astype(jnp.int32)
      yi = y.astype(jnp.int32)
      fi = f.astype(jnp.int32)
      sb = plsc.bitcast(s, jnp.int32)
      valid = e < nreal
      key = jnp.where(fi == 1, C1 - sb, C2 + sb)
      key = jnp.where(valid, key, MAXKEY)
      ck = jnp.where(fi == 1, jnp.int32(0), jnp.int32(4096)) + xi * 64 + yi
      ck = jnp.where(valid, ck, 8192 + (e & 255))
      sl2 = pl.ds(128 * c + 16 * k, 16)
      karr[sl2] = key
      iarr[sl2] = pbase + e
      ckarr[c, sl] = ck
      ordarr[sl2] = ck
      plsc.store_scatter(ptbuf, [e4], x * 16.0)
      plsc.store_scatter(ptbuf, [e4 + 1], y * 16.0)
    return carry

  lax.fori_loop(0, NCH, _s0, jnp.int32(0))
  descs = [pltpu.async_copy(ones128, cnt8k.at[ckarr.at[c]], sem, add=True)
           for c in range(NCH)]
  pltpu.sync_copy(karr, keyA.at[pl.ds(pbase, CH)])
  pltpu.sync_copy(iarr, idxA.at[pl.ds(pbase, CH)])
  pltpu.sync_copy(ordarr, ckey_sh.at[pl.ds(pbase, CH)])
  pltpu.sync_copy(ptbuf, pts_sh.at[pl.ds(pbase * 4, CH * 4)])
  for dsc in descs:
    dsc.wait()
  plsc.subcore_barrier()

  def _one_pass(shift, cur_k, cur_i, nxt_k, nxt_i):
    pltpu.sync_copy(cur_k.at[pl.ds(pbase, CH)], kv)
    pltpu.sync_copy(cur_i.at[pl.ds(pbase, CH)], iv)

    def _zh(i, c):
      hist256[pl.ds(i * 16, 16)] = zeros16
      return c

    lax.fori_loop(0, 16, _zh, jnp.int32(0))

    def _ha(c, carry):
      for k in range(8):
        sl = pl.ds(128 * c + 16 * k, 16)
        kk = kv[sl]
        d = (kk >> shift) & 255
        base = plsc.load_gather(hist256, [d])
        cnt, last = plsc.scan_count(d)
        ordarr[sl] = base + cnt - 1
        plsc.store_scatter(hist256, [d], base + cnt, mask=last)
      return carry

    lax.fori_loop(0, NCH, _ha, jnp.int32(0))
    pltpu.sync_copy(hist256, hist_grid.at[t])
    plsc.subcore_barrier()
    pltpu.sync_copy(hist_grid, grid_l)

    def _bs(c16, carry):
      tot = zeros16
      below = zeros16
      for tt in range(NT):
        row = grid_l[tt, pl.ds(16 * c16, 16)]
        tot = tot + row
        below = below + jnp.where(jnp.int32(tt) < t, row, zeros16)
      csum = plsc.cumsum(tot)
      baserow[pl.ds(16 * c16, 16)] = carry + (csum - tot) + below
      return carry + jnp.sum(tot)

    lax.fori_loop(0, 16, _bs, jnp.int32(0))

    def _sc(c, carry):
      for k in range(8):
        sl = pl.ds(128 * c + 16 * k, 16)
        kk = kv[sl]
        d = (kk >> shift) & 255
        b = plsc.load_gather(baserow, [d])
        posb[c, pl.ds(16 * k, 16)] = b + ordarr[sl]
      return carry

    lax.fori_loop(0, NCH, _sc, jnp.int32(0))
    descs = []
    for c in range(NCH):
      descs.append(pltpu.async_copy(
          kv.at[pl.ds(128 * c, 128)], nxt_k.at[posb.at[c]], sem))
      descs.append(pltpu.async_copy(
          iv.at[pl.ds(128 * c, 128)], nxt_i.at[posb.at[c]], sem))
    for dsc in descs:
      dsc.wait()
    plsc.subcore_barrier()

  def _two_passes(p, carry):
    shift = 16 * p
    _one_pass(shift, keyA, idxA, keyB, idxB)
    _one_pass(shift + 8, keyB, idxB, keyA, idxA)
    return carry

  lax.fori_loop(0, 2, _two_passes, jnp.int32(0))

  pltpu.sync_copy(cnt8k.at[pl.ds(512 * t, 512)], h512)

  def _pr(i, accs):
    pacc, cacc = accs
    hv = h512[pl.ds(16 * i, 16)]
    return pacc + (hv > 0).astype(jnp.int32), cacc + hv

  pacc, cacc = lax.fori_loop(
      0, 32, _pr, (jnp.zeros((16,), jnp.int32), jnp.zeros((16,), jnp.int32)))
  hist256[pl.ds(0, 16)] = pacc
  hist256[pl.ds(16, 16)] = cacc
  pltpu.sync_copy(hist256, stats_grid.at[t])
  plsc.subcore_barrier()
  pltpu.sync_copy(stats_grid, grid_l)
  base_t = jnp.int32(0)
  d_fg = jnp.int32(0)
  n_fg = jnp.int32(0)
  for tt in range(NT):
    rs = jnp.sum(grid_l[tt, pl.ds(0, 16)])
    base_t = base_t + jnp.where(jnp.int32(tt) < t, rs, jnp.int32(0))
    if tt < 8:
      cs = jnp.sum(grid_l[tt, pl.ds(16, 16)])
      d_fg = d_fg + rs
      n_fg = n_fg + cs

  def _rk(i, carry):
    pv = (h512[pl.ds(16 * i, 16)] > 0).astype(jnp.int32)
    cs = plsc.cumsum(pv)
    rk512[pl.ds(16 * i, 16)] = carry + cs - 1
    return carry + jnp.sum(pv)

  lax.fori_loop(0, 32, _rk, base_t)
  pltpu.sync_copy(rk512, rank8k.at[pl.ds(512 * t, 512)])
  plsc.subcore_barrier()

  off_bg = n_fg - d_fg
  obase = RB * t

  descs = [pltpu.async_copy(idxA.at[pl.ds(obase + 128 * c, 128)],
                            sidxb.at[c], sem) for c in range(NCH)]
  for dsc in descs:
    dsc.wait()
  descs = [pltpu.async_copy(ckey_sh.at[sidxb.at[c]], cksb.at[c], sem)
           for c in range(NCH)]
  for dsc in descs:
    dsc.wait()
  descs = [pltpu.async_copy(rank8k.at[cksb.at[c]], rksb.at[c], sem)
           for c in range(NCH)]
  for dsc in descs:
    dsc.wait()

  def _g(c, carry):
    for k in range(8):
      sl = pl.ds(16 * k, 16)
      ck = cksb[c, sl]
      r = rksb[c, sl]
      gb[c, sl] = r + jnp.where(ck >= 4096, off_bg, jnp.int32(0))
    return carry

  lax.fori_loop(0, NCH, _g, jnp.int32(0))
  descs = [pltpu.async_copy(idxA.at[gb.at[c]], jbf.at[pl.ds(128 * c, 128)],
                            sem) for c in range(NCH)]
  for dsc in descs:
    dsc.wait()

  def _x4(c, carry):
    for k in range(32):
      w = 512 * c + 16 * k + lanes
      jv = plsc.load_gather(jbf, [w >> 2])
      idx4[4 * c + k // 8, pl.ds((k % 8) * 16, 16)] = jv * 4 + (w & 3)
    return carry

  lax.fori_loop(0, NCH, _x4, jnp.int32(0))
  descs = [pltpu.async_copy(pts_sh.at[idx4.at[q]],
                            outbuf.at[pl.ds(128 * q, 128)], sem)
           for q in range(4 * NCH)]
  for dsc in descs:
    dsc.wait()
  pltpu.sync_copy(outbuf, out_hbm.at[pl.ds(obase * 4, CH * 4)])


_mesh = plsc.VectorSubcoreMesh(core_axis_name="c", subcore_axis_name="s",
                               num_cores=1)

_sc_call = functools.partial(
    pl.kernel,
    out_type=jax.ShapeDtypeStruct((N * 4,), jnp.float32),
    mesh=_mesh,
    compiler_params=pltpu.CompilerParams(needs_layout_passes=False),
    scratch_types=[
        pltpu.VMEM_SHARED((NPAD,), jnp.int32),
        pltpu.VMEM_SHARED((NPAD,), jnp.int32),
        pltpu.VMEM_SHARED((NPAD,), jnp.int32),
        pltpu.VMEM_SHARED((NPAD,), jnp.int32),
        pltpu.VMEM_SHARED((NPAD,), jnp.int32),
        pltpu.VMEM_SHARED((NPAD * 4,), jnp.float32),
        pltpu.VMEM_SHARED((NT, 256), jnp.int32),
        pltpu.VMEM_SHARED((CNT_BINS,), jnp.int32),
        pltpu.VMEM_SHARED((8192,), jnp.int32),
        pltpu.VMEM_SHARED((NT, 256), jnp.int32),
        pltpu.VMEM((CH * 4,), jnp.float32),
        pltpu.VMEM((CH * 4,), jnp.float32),
        pltpu.VMEM((CH,), jnp.int32),
        pltpu.VMEM((CH,), jnp.int32),
        pltpu.VMEM((NCH, 128), jnp.int32),
        pltpu.VMEM((CH,), jnp.int32),
        pltpu.VMEM((CH,), jnp.int32),
        pltpu.VMEM((CH,), jnp.int32),
        pltpu.VMEM((NCH, 128), jnp.int32),
        pltpu.VMEM((NCH, 128), jnp.int32),
        pltpu.VMEM((NCH, 128), jnp.int32),
        pltpu.VMEM((NCH, 128), jnp.int32),
        pltpu.VMEM((NCH, 128), jnp.int32),
        pltpu.VMEM((CH,), jnp.int32),
        pltpu.VMEM((4 * NCH, 128), jnp.int32),
        pltpu.VMEM((NT, 256), jnp.int32),
        pltpu.VMEM((512,), jnp.int32),
        pltpu.VMEM((512,), jnp.int32),
        pltpu.VMEM((128,), jnp.int32),
        pltpu.VMEM((STRIPE,), jnp.int32),
        pltpu.VMEM((256,), jnp.int32),
        pltpu.VMEM((256,), jnp.int32),
        pltpu.SemaphoreType.DMA,
    ],
)(_body)


@jax.jit
def kernel(points):
  flat = jnp.reshape(points.astype(jnp.float32), (N * 4,))
  return jnp.reshape(_sc_call(flat), (N, 4))


# speedup vs baseline: 3.3448x; 1.0097x over previous
import functools

import jax
import jax.numpy as jnp
from jax import lax
from jax.experimental import pallas as pl
from jax.experimental.pallas import tpu as pltpu
from jax.experimental.pallas import tpu_sc as plsc

N = 20000
NT = 16
CH = 1280
NPAD = NT * CH
RB = 1248
NCH = 10
MAXKEY = 0x7FFFFFFF
C1 = (1 << 30) - 1
C2 = 1 << 30
CNT_BINS = 8448
STRIPE = CNT_BINS // NT


def _body(pts_hbm, out_hbm,
          keyA, idxA, keyB, idxB, ckey_sh, pts_sh, hist_grid, cnt8k, rank8k,
          stats_grid,
          ptbuf, outbuf, karr, iarr, ckarr, ordarr, kv, iv, posb, sidxb,
          cksb, rksb, gb, jbf, idx4, grid_l, h512, rk512,
          ones128, zstripe, hist256, baserow, sem):
  t = lax.axis_index("s")
  lanes = lax.iota(jnp.int32, 16)
  zeros16 = jnp.zeros((16,), jnp.int32)
  col1 = jnp.full((16,), 1, jnp.int32)
  col2 = jnp.full((16,), 2, jnp.int32)
  col3 = jnp.full((16,), 3, jnp.int32)

  base_row = RB * t
  pbase = CH * t
  nreal = jnp.where(t == NT - 1, jnp.int32(CH), jnp.int32(RB))

  pltpu.sync_copy(pts_hbm.at[pl.ds(base_row * 4, CH * 4)], ptbuf)

  def _ob(i, c):
    ones128[pl.ds(i * 16, 16)] = jnp.full((16,), 1, jnp.int32)
    return c

  lax.fori_loop(0, 8, _ob, jnp.int32(0))

  def _zb(i, c):
    zstripe[pl.ds(i * 16, 16)] = zeros16
    return c

  lax.fori_loop(0, STRIPE // 16, _zb, jnp.int32(0))
  pltpu.sync_copy(zstripe, cnt8k.at[pl.ds(STRIPE * t, STRIPE)])
  plsc.subcore_barrier()

  def _s0(c, carry):
    for k in range(8):
      sl = pl.ds(16 * k, 16)
      e = 128 * c + 16 * k + lanes
      e4 = e * 4
      x = plsc.load_gather(ptbuf, [e4])
      y = plsc.load_gather(ptbuf, [e4 + 1])
      s = plsc.load_gather(ptbuf, [e4 + 2])
      f = plsc.load_gather(ptbuf, [e4 + 3])
      xi = x.astype(jnp.int32)
      yi = y.astype(jnp.int32)
      fi = f.astype(jnp.int32)
      sb = plsc.bitcast(s, jnp.int32)
      valid = e < nreal
      key = jnp.where(fi == 1, C1 - sb, C2 + sb)
      key = jnp.where(valid, key, MAXKEY)
      ck = jnp.where(fi == 1, jnp.int32(0), jnp.int32(4096)) + xi * 64 + yi
      ck = jnp.where(valid, ck, 8192 + (e & 255))
      sl2 = pl.ds(128 * c + 16 * k, 16)
      karr[sl2] = key
      iarr[sl2] = pbase + e
      ckarr[c, sl] = ck
      ordarr[sl2] = ck
      plsc.store_scatter(ptbuf, [e4], x * 16.0)
      plsc.store_scatter(ptbuf, [e4 + 1], y * 16.0)
    return carry

  lax.fori_loop(0, NCH, _s0, jnp.int32(0))
  descs = [pltpu.async_copy(ones128, cnt8k.at[ckarr.at[c]], sem, add=True)
           for c in range(NCH)]
  pltpu.sync_copy(ordarr, ckey_sh.at[pl.ds(pbase, CH)])
  pltpu.sync_copy(ptbuf, pts_sh.at[pl.ds(pbase * 4, CH * 4)])
  for dsc in descs:
    dsc.wait()

  def _hist_and_pos(vals_ref, dig):
    def _zh(i, c):
      hist256[pl.ds(i * 16, 16)] = zeros16
      return c

    lax.fori_loop(0, 16, _zh, jnp.int32(0))

    def _ha(c, carry):
      for k in range(8):
        sl = pl.ds(128 * c + 16 * k, 16)
        d = dig(vals_ref[sl])
        base = plsc.load_gather(hist256, [d])
        cnt, last = plsc.scan_count(d)
        ordarr[sl] = base + cnt - 1
        plsc.store_scatter(hist256, [d], base + cnt, mask=last)
      return carry

    lax.fori_loop(0, NCH, _ha, jnp.int32(0))
    pltpu.sync_copy(hist256, hist_grid.at[t])
    plsc.subcore_barrier()
    pltpu.sync_copy(hist_grid, grid_l)

    def _bs(c16, carry):
      tot = zeros16
      below = zeros16
      for tt in range(NT):
        row = grid_l[tt, pl.ds(16 * c16, 16)]
        tot = tot + row
        below = below + jnp.where(jnp.int32(tt) < t, row, zeros16)
      csum = plsc.cumsum(tot)
      baserow[pl.ds(16 * c16, 16)] = carry + (csum - tot) + below
      return carry + jnp.sum(tot)

    lax.fori_loop(0, 16, _bs, jnp.int32(0))

    def _sc(c, carry):
      for k in range(8):
        sl = pl.ds(128 * c + 16 * k, 16)
        d = dig(vals_ref[sl])
        b = plsc.load_gather(baserow, [d])
        posb[c, pl.ds(16 * k, 16)] = b + ordarr[sl]
      return carry

    lax.fori_loop(0, NCH, _sc, jnp.int32(0))

  def _fire_scatter(srcs_dsts):
    descs = []
    for c in range(NCH):
      for src, dst in srcs_dsts:
        descs.append(pltpu.async_copy(
            src.at[pl.ds(128 * c, 128)], dst.at[posb.at[c]], sem))
    for dsc in descs:
      dsc.wait()
    plsc.subcore_barrier()

  _hist_and_pos(karr, lambda v: v & 255)
  _fire_scatter([(karr, keyB), (iarr, idxB)])

  pltpu.sync_copy(keyB.at[pl.ds(pbase, CH)], kv)
  pltpu.sync_copy(idxB.at[pl.ds(pbase, CH)], iv)
  _hist_and_pos(kv, lambda v: (v >> 8) & 255)

  def _pk(c, carry):
    for k in range(8):
      sl = pl.ds(128 * c + 16 * k, 16)
      karr[sl] = ((kv[sl] >> 16) << 15) | iv[sl]
    return carry

  lax.fori_loop(0, NCH, _pk, jnp.int32(0))
  _fire_scatter([(karr, keyA)])

  pltpu.sync_copy(keyA.at[pl.ds(pbase, CH)], kv)
  _hist_and_pos(kv, lambda v: (v >> 15) & 255)
  _fire_scatter([(kv, idxA)])

  pltpu.sync_copy(idxA.at[pl.ds(pbase, CH)], kv)
  _hist_and_pos(kv, lambda v: (v >> 23) & 255)
  _fire_scatter([(kv, keyA)])

  pltpu.sync_copy(cnt8k.at[pl.ds(512 * t, 512)], h512)

  def _pr(i, accs):
    pacc, cacc = accs
    hv = h512[pl.ds(16 * i, 16)]
    return pacc + (hv > 0).astype(jnp.int32), cacc + hv

  pacc, cacc = lax.fori_loop(
      0, 32, _pr, (jnp.zeros((16,), jnp.int32), jnp.zeros((16,), jnp.int32)))
  hist256[pl.ds(0, 16)] = pacc
  hist256[pl.ds(16, 16)] = cacc
  pltpu.sync_copy(hist256, stats_grid.at[t])
  plsc.subcore_barrier()
  pltpu.sync_copy(stats_grid, grid_l)
  base_t = jnp.int32(0)
  d_fg = jnp.int32(0)
  n_fg = jnp.int32(0)
  for tt in range(NT):
    rs = jnp.sum(grid_l[tt, pl.ds(0, 16)])
    base_t = base_t + jnp.where(jnp.int32(tt) < t, rs, jnp.int32(0))
    if tt < 8:
      cs = jnp.sum(grid_l[tt, pl.ds(16, 16)])
      d_fg = d_fg + rs
      n_fg = n_fg + cs

  def _rk(i, carry):
    pv = (h512[pl.ds(16 * i, 16)] > 0).astype(jnp.int32)
    cs = plsc.cumsum(pv)
    rk512[pl.ds(16 * i, 16)] = carry + cs - 1
    return carry + jnp.sum(pv)

  lax.fori_loop(0, 32, _rk, base_t)
  pltpu.sync_copy(rk512, rank8k.at[pl.ds(512 * t, 512)])
  plsc.subcore_barrier()

  off_bg = n_fg - d_fg
  obase = RB * t

  descs = [pltpu.async_copy(keyA.at[pl.ds(obase + 128 * c, 128)],
                            sidxb.at[c], sem) for c in range(NCH)]
  for dsc in descs:
    dsc.wait()

  def _msk(c, carry):
    for k in range(8):
      sl = pl.ds(16 * k, 16)
      sidxb[c, sl] = sidxb[c, sl] & 0x7FFF
    return carry

  lax.fori_loop(0, NCH, _msk, jnp.int32(0))
  descs = [pltpu.async_copy(ckey_sh.at[sidxb.at[c]], cksb.at[c], sem)
           for c in range(NCH)]
  for dsc in descs:
    dsc.wait()
  descs = [pltpu.async_copy(rank8k.at[cksb.at[c]], rksb.at[c], sem)
           for c in range(NCH)]
  for dsc in descs:
    dsc.wait()

  def _g(c, carry):
    for k in range(8):
      sl = pl.ds(16 * k, 16)
      ck = cksb[c, sl]
      r = rksb[c, sl]
      gb[c, sl] = r + jnp.where(ck >= 4096, off_bg, jnp.int32(0))
    return carry

  lax.fori_loop(0, NCH, _g, jnp.int32(0))
  descs = [pltpu.async_copy(keyA.at[gb.at[c]], jbf.at[pl.ds(128 * c, 128)],
                            sem) for c in range(NCH)]
  for dsc in descs:
    dsc.wait()

  def _x4(c, carry):
    for k in range(32):
      w = 512 * c + 16 * k + lanes
      jv = plsc.load_gather(jbf, [w >> 2]) & 0x7FFF
      idx4[4 * c + k // 8, pl.ds((k % 8) * 16, 16)] = jv * 4 + (w & 3)
    return carry

  lax.fori_loop(0, NCH, _x4, jnp.int32(0))
  descs = [pltpu.async_copy(pts_sh.at[idx4.at[q]],
                            outbuf.at[pl.ds(128 * q, 128)], sem)
           for q in range(4 * NCH)]
  for dsc in descs:
    dsc.wait()
  pltpu.sync_copy(outbuf, out_hbm.at[pl.ds(obase * 4, CH * 4)])


_mesh = plsc.VectorSubcoreMesh(core_axis_name="c", subcore_axis_name="s",
                               num_cores=1)

_sc_call = functools.partial(
    pl.kernel,
    out_type=jax.ShapeDtypeStruct((N * 4,), jnp.float32),
    mesh=_mesh,
    compiler_params=pltpu.CompilerParams(needs_layout_passes=False),
    scratch_types=[
        pltpu.VMEM_SHARED((NPAD,), jnp.int32),
        pltpu.VMEM_SHARED((NPAD,), jnp.int32),
        pltpu.VMEM_SHARED((NPAD,), jnp.int32),
        pltpu.VMEM_SHARED((NPAD,), jnp.int32),
        pltpu.VMEM_SHARED((NPAD,), jnp.int32),
        pltpu.VMEM_SHARED((NPAD * 4,), jnp.float32),
        pltpu.VMEM_SHARED((NT, 256), jnp.int32),
        pltpu.VMEM_SHARED((CNT_BINS,), jnp.int32),
        pltpu.VMEM_SHARED((8192,), jnp.int32),
        pltpu.VMEM_SHARED((NT, 256), jnp.int32),
        pltpu.VMEM((CH * 4,), jnp.float32),
        pltpu.VMEM((CH * 4,), jnp.float32),
        pltpu.VMEM((CH,), jnp.int32),
        pltpu.VMEM((CH,), jnp.int32),
        pltpu.VMEM((NCH, 128), jnp.int32),
        pltpu.VMEM((CH,), jnp.int32),
        pltpu.VMEM((CH,), jnp.int32),
        pltpu.VMEM((CH,), jnp.int32),
        pltpu.VMEM((NCH, 128), jnp.int32),
        pltpu.VMEM((NCH, 128), jnp.int32),
        pltpu.VMEM((NCH, 128), jnp.int32),
        pltpu.VMEM((NCH, 128), jnp.int32),
        pltpu.VMEM((NCH, 128), jnp.int32),
        pltpu.VMEM((CH,), jnp.int32),
        pltpu.VMEM((4 * NCH, 128), jnp.int32),
        pltpu.VMEM((NT, 256), jnp.int32),
        pltpu.VMEM((512,), jnp.int32),
        pltpu.VMEM((512,), jnp.int32),
        pltpu.VMEM((128,), jnp.int32),
        pltpu.VMEM((STRIPE,), jnp.int32),
        pltpu.VMEM((256,), jnp.int32),
        pltpu.VMEM((256,), jnp.int32),
        pltpu.SemaphoreType.DMA,
    ],
)(_body)


@jax.jit
def kernel(points):
  flat = jnp.reshape(points.astype(jnp.float32), (N * 4,))
  return jnp.reshape(_sc_call(flat), (N, 4))


# speedup vs baseline: 3.3533x; 1.0025x over previous
import functools

import jax
import jax.numpy as jnp
from jax import lax
from jax.experimental import pallas as pl
from jax.experimental.pallas import tpu as pltpu
from jax.experimental.pallas import tpu_sc as plsc

N = 20000
NT = 16
CH = 1280
NPAD = NT * CH
RB = 1248
NCH = 10
MAXKEY = 0x7FFFFFFF
C1 = (1 << 30) - 1
C2 = 1 << 30
CNT_BINS = 8448
STRIPE = CNT_BINS // NT


def _body(pts_hbm, out_hbm,
          keyA, idxA, keyB, idxB, ckey_sh, pts_sh, hist_grid, cnt8k, rank8k,
          stats_grid,
          ptbuf, outbuf, karr, iarr, ckarr, ordarr, kv, iv, posb, sidxb,
          cksb, rksb, gb, jbf, idx4, grid_l, h512, rk512,
          ones128, zstripe, hist256, baserow, sem):
  t = lax.axis_index("s")
  lanes = lax.iota(jnp.int32, 16)
  zeros16 = jnp.zeros((16,), jnp.int32)

  base_row = RB * t
  pbase = CH * t
  nreal = jnp.where(t == NT - 1, jnp.int32(CH), jnp.int32(RB))

  pltpu.sync_copy(pts_hbm.at[pl.ds(base_row * 4, CH * 4)], ptbuf)

  def _ob(i, c):
    ones128[pl.ds(i * 16, 16)] = jnp.full((16,), 1, jnp.int32)
    return c

  lax.fori_loop(0, 8, _ob, jnp.int32(0))

  def _zb(i, c):
    zstripe[pl.ds(i * 16, 16)] = zeros16
    return c

  lax.fori_loop(0, STRIPE // 16, _zb, jnp.int32(0))
  pltpu.sync_copy(zstripe, cnt8k.at[pl.ds(STRIPE * t, STRIPE)])
  plsc.subcore_barrier()

  def _s0(c, carry):
    for k in range(8):
      sl = pl.ds(16 * k, 16)
      e = 128 * c + 16 * k + lanes
      e4 = e * 4
      x = plsc.load_gather(ptbuf, [e4])
      y = plsc.load_gather(ptbuf, [e4 + 1])
      s = plsc.load_gather(ptbuf, [e4 + 2])
      f = plsc.load_gather(ptbuf, [e4 + 3])
      xi = x.astype(jnp.int32)
      yi = y.astype(jnp.int32)
      fi = f.astype(jnp.int32)
      sb = plsc.bitcast(s, jnp.int32)
      valid = e < nreal
      key = jnp.where(fi == 1, C1 - sb, C2 + sb)
      key = jnp.where(valid, key, MAXKEY)
      ck = jnp.where(fi == 1, jnp.int32(0), jnp.int32(4096)) + xi * 64 + yi
      ck = jnp.where(valid, ck, 8192 + (e & 255))
      sl2 = pl.ds(128 * c + 16 * k, 16)
      karr[sl2] = key
      iarr[sl2] = pbase + e
      ckarr[c, sl] = ck
      ordarr[sl2] = ck
      plsc.store_scatter(ptbuf, [e4], x * 16.0)
      plsc.store_scatter(ptbuf, [e4 + 1], y * 16.0)
    return carry

  lax.fori_loop(0, NCH, _s0, jnp.int32(0))
  descs = [pltpu.async_copy(ones128, cnt8k.at[ckarr.at[c]], sem, add=True)
           for c in range(NCH)]
  pltpu.sync_copy(ordarr, ckey_sh.at[pl.ds(pbase, CH)])
  pltpu.sync_copy(ptbuf, pts_sh.at[pl.ds(pbase * 4, CH * 4)])
  for dsc in descs:
    dsc.wait()

  def _hist_and_pos(vals_ref, dig):
    def _zh(i, c):
      hist256[pl.ds(i * 16, 16)] = zeros16
      return c

    lax.fori_loop(0, 16, _zh, jnp.int32(0))

    def _ha(c, carry):
      for k in range(8):
        sl = pl.ds(128 * c + 16 * k, 16)
        d = dig(vals_ref[sl])
        base = plsc.load_gather(hist256, [d])
        cnt, last = plsc.scan_count(d)
        ordarr[sl] = base + cnt - 1
        plsc.store_scatter(hist256, [d], base + cnt, mask=last)
      return carry

    lax.fori_loop(0, NCH, _ha, jnp.int32(0))
    pltpu.sync_copy(hist256, hist_grid.at[t])
    plsc.subcore_barrier()
    pltpu.sync_copy(hist_grid, grid_l)

    def _bs(c16, carry):
      tot = zeros16
      below = zeros16
      for tt in range(NT):
        row = grid_l[tt, pl.ds(16 * c16, 16)]
        tot = tot + row
        below = below + jnp.where(jnp.int32(tt) < t, row, zeros16)
      csum = plsc.cumsum(tot)
      baserow[pl.ds(16 * c16, 16)] = carry + (csum - tot) + below
      return carry + jnp.sum(tot)

    lax.fori_loop(0, 16, _bs, jnp.int32(0))

    def _sc(c, carry):
      for k in range(8):
        sl = pl.ds(128 * c + 16 * k, 16)
        d = dig(vals_ref[sl])
        b = plsc.load_gather(baserow, [d])
        posb[c, pl.ds(16 * k, 16)] = b + ordarr[sl]
      return carry

    lax.fori_loop(0, NCH, _sc, jnp.int32(0))

  def _fire_scatter(srcs_dsts):
    descs = []
    for c in range(NCH):
      for src, dst in srcs_dsts:
        descs.append(pltpu.async_copy(
            src.at[pl.ds(128 * c, 128)], dst.at[posb.at[c]], sem))
    for dsc in descs:
      dsc.wait()
    plsc.subcore_barrier()

  _hist_and_pos(karr, lambda v: v & 255)
  _fire_scatter([(karr, keyB), (iarr, idxB)])

  pltpu.sync_copy(keyB.at[pl.ds(pbase, CH)], kv)
  pltpu.sync_copy(idxB.at[pl.ds(pbase, CH)], iv)
  _hist_and_pos(kv, lambda v: (v >> 8) & 255)

  def _pk(c, carry):
    for k in range(8):
      sl = pl.ds(128 * c + 16 * k, 16)
      karr[sl] = ((kv[sl] >> 16) << 15) | iv[sl]
    return carry

  lax.fori_loop(0, NCH, _pk, jnp.int32(0))
  _fire_scatter([(karr, keyA)])

  pltpu.sync_copy(cnt8k.at[pl.ds(512 * t, 512)], h512)

  def _pr(i, accs):
    pacc, cacc = accs
    hv = h512[pl.ds(16 * i, 16)]
    return pacc + (hv > 0).astype(jnp.int32), cacc + hv

  pacc, cacc = lax.fori_loop(
      0, 32, _pr, (jnp.zeros((16,), jnp.int32), jnp.zeros((16,), jnp.int32)))
  rk512[pl.ds(0, 16)] = pacc
  rk512[pl.ds(16, 16)] = cacc
  pltpu.sync_copy(rk512.at[pl.ds(0, 256)], stats_grid.at[t])

  pltpu.sync_copy(keyA.at[pl.ds(pbase, CH)], kv)
  _hist_and_pos(kv, lambda v: (v >> 15) & 255)
  _fire_scatter([(kv, idxA)])

  pltpu.sync_copy(idxA.at[pl.ds(pbase, CH)], kv)
  _hist_and_pos(kv, lambda v: (v >> 23) & 255)
  _fire_scatter([(kv, keyA)])

  pltpu.sync_copy(stats_grid, grid_l)
  base_t = jnp.int32(0)
  d_fg = jnp.int32(0)
  n_fg = jnp.int32(0)
  for tt in range(NT):
    rs = jnp.sum(grid_l[tt, pl.ds(0, 16)])
    base_t = base_t + jnp.where(jnp.int32(tt) < t, rs, jnp.int32(0))
    if tt < 8:
      cs = jnp.sum(grid_l[tt, pl.ds(16, 16)])
      d_fg = d_fg + rs
      n_fg = n_fg + cs

  def _rk(i, carry):
    pv = (h512[pl.ds(16 * i, 16)] > 0).astype(jnp.int32)
    cs = plsc.cumsum(pv)
    rk512[pl.ds(16 * i, 16)] = carry + cs - 1
    return carry + jnp.sum(pv)

  lax.fori_loop(0, 32, _rk, base_t)
  pltpu.sync_copy(rk512, rank8k.at[pl.ds(512 * t, 512)])
  plsc.subcore_barrier()

  off_bg = n_fg - d_fg
  obase = RB * t

  descs = [pltpu.async_copy(keyA.at[pl.ds(obase + 128 * c, 128)],
                            sidxb.at[c], sem) for c in range(NCH)]
  for dsc in descs:
    dsc.wait()

  def _msk(c, carry):
    for k in range(8):
      sl = pl.ds(16 * k, 16)
      sidxb[c, sl] = sidxb[c, sl] & 0x7FFF
    return carry

  lax.fori_loop(0, NCH, _msk, jnp.int32(0))
  descs = [pltpu.async_copy(ckey_sh.at[sidxb.at[c]], cksb.at[c], sem)
           for c in range(NCH)]
  for dsc in descs:
    dsc.wait()
  descs = [pltpu.async_copy(rank8k.at[cksb.at[c]], rksb.at[c], sem)
           for c in range(NCH)]
  for dsc in descs:
    dsc.wait()

  def _g(c, carry):
    for k in range(8):
      sl = pl.ds(16 * k, 16)
      ck = cksb[c, sl]
      r = rksb[c, sl]
      gb[c, sl] = r + jnp.where(ck >= 4096, off_bg, jnp.int32(0))
    return carry

  lax.fori_loop(0, NCH, _g, jnp.int32(0))
  descs = [pltpu.async_copy(keyA.at[gb.at[c]], jbf.at[pl.ds(128 * c, 128)],
                            sem) for c in range(NCH)]
  for dsc in descs:
    dsc.wait()

  def _x4(c, carry):
    for k in range(32):
      w = 512 * c + 16 * k + lanes
      jv = plsc.load_gather(jbf, [w >> 2]) & 0x7FFF
      idx4[4 * c + k // 8, pl.ds((k % 8) * 16, 16)] = jv * 4 + (w & 3)
    return carry

  lax.fori_loop(0, NCH, _x4, jnp.int32(0))
  descs = [pltpu.async_copy(pts_sh.at[idx4.at[q]],
                            outbuf.at[pl.ds(128 * q, 128)], sem)
           for q in range(4 * NCH)]
  for dsc in descs:
    dsc.wait()
  pltpu.sync_copy(outbuf, out_hbm.at[pl.ds(obase * 4, CH * 4)])


_mesh = plsc.VectorSubcoreMesh(core_axis_name="c", subcore_axis_name="s",
                               num_cores=1)

_sc_call = functools.partial(
    pl.kernel,
    out_type=jax.ShapeDtypeStruct((N * 4,), jnp.float32),
    mesh=_mesh,
    compiler_params=pltpu.CompilerParams(needs_layout_passes=False),
    scratch_types=[
        pltpu.VMEM_SHARED((NPAD,), jnp.int32),
        pltpu.VMEM_SHARED((NPAD,), jnp.int32),
        pltpu.VMEM_SHARED((NPAD,), jnp.int32),
        pltpu.VMEM_SHARED((NPAD,), jnp.int32),
        pltpu.VMEM_SHARED((NPAD,), jnp.int32),
        pltpu.VMEM_SHARED((NPAD * 4,), jnp.float32),
        pltpu.VMEM_SHARED((NT, 256), jnp.int32),
        pltpu.VMEM_SHARED((CNT_BINS,), jnp.int32),
        pltpu.VMEM_SHARED((8192,), jnp.int32),
        pltpu.VMEM_SHARED((NT, 256), jnp.int32),
        pltpu.VMEM((CH * 4,), jnp.float32),
        pltpu.VMEM((CH * 4,), jnp.float32),
        pltpu.VMEM((CH,), jnp.int32),
        pltpu.VMEM((CH,), jnp.int32),
        pltpu.VMEM((NCH, 128), jnp.int32),
        pltpu.VMEM((CH,), jnp.int32),
        pltpu.VMEM((CH,), jnp.int32),
        pltpu.VMEM((CH,), jnp.int32),
        pltpu.VMEM((NCH, 128), jnp.int32),
        pltpu.VMEM((NCH, 128), jnp.int32),
        pltpu.VMEM((NCH, 128), jnp.int32),
        pltpu.VMEM((NCH, 128), jnp.int32),
        pltpu.VMEM((NCH, 128), jnp.int32),
        pltpu.VMEM((CH,), jnp.int32),
        pltpu.VMEM((4 * NCH, 128), jnp.int32),
        pltpu.VMEM((NT, 256), jnp.int32),
        pltpu.VMEM((512,), jnp.int32),
        pltpu.VMEM((512,), jnp.int32),
        pltpu.VMEM((128,), jnp.int32),
        pltpu.VMEM((STRIPE,), jnp.int32),
        pltpu.VMEM((256,), jnp.int32),
        pltpu.VMEM((256,), jnp.int32),
        pltpu.SemaphoreType.DMA,
    ],
)(_body)


@jax.jit
def kernel(points):
  flat = jnp.reshape(points.astype(jnp.float32), (N * 4,))
  return jnp.reshape(_sc_call(flat), (N, 4))
